# baseline (speedup 1.0000x reference)
import jax
import jax.numpy as jnp
from jax import lax
from jax.experimental import pallas as pl
from jax.experimental.pallas import tpu as pltpu
from jax.experimental.pallas import tpu_sc as plsc

E = 320000
D = 128
NC, NS = 2, 16
NW = NC * NS
NB = E // 128
BPW = NB // NW
XTRA = NB % NW
CHUNK = 384
NCHUNK = (BPW * 128) // CHUNK
SUB = 128
NSUB = CHUNK // SUB
NGRP = CHUNK // 16
GPS = SUB // 16
NROW = 64


def _edge_encoder_body(ef_hbm, w0_hbm, w1_hbm, w2_hbm, out_hbm,
                       w0_v, w1_v, w2_v, combo_v, combo_sp, ef_v0, ef_v1,
                       codes0, codes1, rows0, rows1,
                       efsem, gsem0, gsem1, ssem0, ssem1):
    efb = (ef_v0, ef_v1)
    codes = (codes0, codes1)
    rows = (rows0, rows1)
    gsem = (gsem0, gsem1)
    ssem = (ssem0, ssem1)
    wid = lax.axis_index("s") * NC + lax.axis_index("c")

    pltpu.sync_copy(w0_hbm, w0_v)
    pltpu.sync_copy(w1_hbm, w1_v)
    pltpu.sync_copy(w2_hbm, w2_v)
    zero = jnp.zeros((16,), jnp.float32)
    for i0 in range(5):
        for c in range(8):
            s = pl.ds(c * 16, 16)
            t0 = w0_v[i0, s]
            for i1 in range(6):
                t01 = t0 + w1_v[i1, s]
                for i2 in range(2):
                    combo_v[(i0 * 6 + i1) * 2 + i2, s] = t01 + w2_v[i2, s]
    for r in range(60, NROW):
        for c in range(8):
            combo_v[r, pl.ds(c * 16, 16)] = zero
    @pl.when(lax.axis_index("s") == 0)
    def _publish():
        pltpu.sync_copy(combo_v, combo_sp)

    plsc.subcore_barrier()

    wbase = 128 * (BPW * wid + jnp.minimum(wid, XTRA))

    def fire_ef(ci, b):
        eoff = pl.multiple_of(wbase + ci * CHUNK, 128)
        pltpu.async_copy(ef_hbm.at[:, pl.ds(eoff, CHUNK)], efb[b], efsem)

    def drain_ef(b):
        pltpu.make_async_copy(
            ef_hbm.at[:, pl.ds(0, CHUNK)], efb[b], efsem).wait()

    def compute_codes(eb, b):
        for g in range(NGRP):
            s16 = pl.ds(g * 16, 16)
            i0 = efb[eb][0, s16]
            i1 = efb[eb][1, s16]
            i2 = efb[eb][2, s16]
            codes[b][g // GPS, pl.ds((g % GPS) * 16, 16)] = (
                (i0 * 6 + i1) * 2 + i2)

    def fire_gathers(b):
        for j in range(NSUB):
            pltpu.async_copy(combo_sp.at[codes[b].at[j]],
                             rows[b].at[pl.ds(j * SUB, SUB), :], gsem[b])

    def drain_gathers(b):
        pltpu.make_async_copy(
            out_hbm.at[pl.ds(0, CHUNK), :], rows[b], gsem[b]).wait()

    def fire_scatter(ci, b):
        start = pl.multiple_of(wbase + ci * CHUNK, 128)
        pltpu.async_copy(rows[b], out_hbm.at[pl.ds(start, CHUNK), :], ssem[b])

    def drain_scatter(b):
        pltpu.make_async_copy(
            rows[b], out_hbm.at[pl.ds(0, CHUNK), :], ssem[b]).wait()

    fire_ef(0, 0)
    drain_ef(0)
    compute_codes(0, 0)
    fire_ef(1, 1)
    fire_gathers(0)
    drain_ef(1)
    fire_ef(2, 0)
    compute_codes(1, 1)
    fire_gathers(1)
    drain_gathers(0)
    fire_scatter(0, 0)

    def pair_body(i, carry):
        for k in (0, 1):
            ci = 2 * i + 1 + k
            cur = (1 + k) % 2
            nxt = 1 - cur
            drain_ef(nxt)
            compute_codes(nxt, nxt)

            @pl.when(ci + 2 < NCHUNK)
            def _prefetch():
                fire_ef(ci + 2, cur)

            drain_scatter(nxt)
            fire_gathers(nxt)
            drain_gathers(cur)
            fire_scatter(ci, cur)
        return carry

    lax.fori_loop(0, (NCHUNK - 2) // 2, pair_body, 0)
    drain_scatter(0)
    drain_gathers(1)
    fire_scatter(NCHUNK - 1, 1)
    drain_scatter(1)

    @pl.when(wid < XTRA)
    def _extra():
        estart = pl.multiple_of(wbase + BPW * 128, 128)
        pltpu.async_copy(ef_hbm.at[:, pl.ds(estart, 128)],
                         efb[0].at[:, pl.ds(0, 128)], efsem)
        pltpu.make_async_copy(
            ef_hbm.at[:, pl.ds(0, 128)],
            efb[0].at[:, pl.ds(0, 128)], efsem).wait()
        zero_i = jnp.zeros((16,), jnp.int32)
        for g in range(8):
            s16 = pl.ds(g * 16, 16)
            i0 = efb[0][0, s16]
            i1 = efb[0][1, s16]
            i2 = efb[0][2, s16]
            codes[0][g // GPS, pl.ds((g % GPS) * 16, 16)] = (
                (i0 * 6 + i1) * 2 + i2)
        for g in range(8, 2 * GPS):
            codes[0][g // GPS, pl.ds((g % GPS) * 16, 16)] = zero_i
        for j in range(2):
            pltpu.async_copy(combo_sp.at[codes[0].at[j]],
                             rows[0].at[pl.ds(j * SUB, SUB), :], gsem[0])
        pltpu.make_async_copy(
            out_hbm.at[pl.ds(0, 2 * SUB), :],
            rows[0].at[pl.ds(0, 2 * SUB), :], gsem[0]).wait()
        pltpu.sync_copy(rows[0].at[pl.ds(0, 128), :],
                        out_hbm.at[pl.ds(estart, 128), :])


def kernel(edge_feature, W0, W1, W2):
    ef_packed = edge_feature.T
    mesh = plsc.VectorSubcoreMesh(core_axis_name="c", subcore_axis_name="s")
    f = pl.kernel(
        _edge_encoder_body,
        out_type=jax.ShapeDtypeStruct((E, D), jnp.float32),
        mesh=mesh,
        scratch_types=[
            pltpu.VMEM((5, D), jnp.float32),
            pltpu.VMEM((6, D), jnp.float32),
            pltpu.VMEM((2, D), jnp.float32),
            pltpu.VMEM((NROW, D), jnp.float32),
            pltpu.VMEM_SHARED((NROW, D), jnp.float32),
            pltpu.VMEM((3, CHUNK), jnp.int32),
            pltpu.VMEM((3, CHUNK), jnp.int32),
            pltpu.VMEM((NSUB, SUB), jnp.int32),
            pltpu.VMEM((NSUB, SUB), jnp.int32),
            pltpu.VMEM((CHUNK, D), jnp.float32),
            pltpu.VMEM((CHUNK, D), jnp.float32),
            pltpu.SemaphoreType.DMA,
            pltpu.SemaphoreType.DMA,
            pltpu.SemaphoreType.DMA,
            pltpu.SemaphoreType.DMA,
            pltpu.SemaphoreType.DMA,
        ],
    )
    return f(ef_packed, W0, W1, W2)


# speedup vs baseline: 1.0051x; 1.0051x over previous
import jax
import jax.numpy as jnp
from jax import lax
from jax.experimental import pallas as pl
from jax.experimental.pallas import tpu as pltpu
from jax.experimental.pallas import tpu_sc as plsc

E = 320000
D = 128
NC, NS = 2, 16
NW = NC * NS
NB = E // 128
BPW = NB // NW
XTRA = NB % NW
CHUNK = 384
NCHUNK = (BPW * 128) // CHUNK
SUB = 96
NSUB = CHUNK // SUB
NGRP = CHUNK // 16
GPS = SUB // 16
NROW = 64


def _edge_encoder_body(ef_hbm, w0_hbm, w1_hbm, w2_hbm, out_hbm,
                       w0_v, w1_v, w2_v, combo_v, combo_sp, ef_v0, ef_v1,
                       codes0, codes1, rows0, rows1,
                       efsem, gsem0, gsem1, ssem0, ssem1):
    efb = (ef_v0, ef_v1)
    codes = (codes0, codes1)
    rows = (rows0, rows1)
    gsem = (gsem0, gsem1)
    ssem = (ssem0, ssem1)
    wid = lax.axis_index("s") * NC + lax.axis_index("c")

    pltpu.sync_copy(w0_hbm, w0_v)
    pltpu.sync_copy(w1_hbm, w1_v)
    pltpu.sync_copy(w2_hbm, w2_v)
    zero = jnp.zeros((16,), jnp.float32)
    for i0 in range(5):
        for c in range(8):
            s = pl.ds(c * 16, 16)
            t0 = w0_v[i0, s]
            for i1 in range(6):
                t01 = t0 + w1_v[i1, s]
                for i2 in range(2):
                    combo_v[(i0 * 6 + i1) * 2 + i2, s] = t01 + w2_v[i2, s]
    for r in range(60, NROW):
        for c in range(8):
            combo_v[r, pl.ds(c * 16, 16)] = zero
    @pl.when(lax.axis_index("s") == 0)
    def _publish():
        pltpu.sync_copy(combo_v, combo_sp)

    plsc.subcore_barrier()

    wbase = 128 * (BPW * wid + jnp.minimum(wid, XTRA))

    def fire_ef(ci, b):
        eoff = pl.multiple_of(wbase + ci * CHUNK, 128)
        pltpu.async_copy(ef_hbm.at[:, pl.ds(eoff, CHUNK)], efb[b], efsem)

    def drain_ef(b):
        pltpu.make_async_copy(
            ef_hbm.at[:, pl.ds(0, CHUNK)], efb[b], efsem).wait()

    def compute_codes(eb, b):
        for g in range(NGRP):
            s16 = pl.ds(g * 16, 16)
            i0 = efb[eb][0, s16]
            i1 = efb[eb][1, s16]
            i2 = efb[eb][2, s16]
            codes[b][g // GPS, pl.ds((g % GPS) * 16, 16)] = (
                (i0 * 6 + i1) * 2 + i2)

    def fire_gathers(b):
        for j in range(NSUB):
            pltpu.async_copy(combo_sp.at[codes[b].at[j]],
                             rows[b].at[pl.ds(j * SUB, SUB), :], gsem[b])

    def drain_gathers(b):
        pltpu.make_async_copy(
            out_hbm.at[pl.ds(0, CHUNK), :], rows[b], gsem[b]).wait()

    def fire_scatter(ci, b):
        start = pl.multiple_of(wbase + ci * CHUNK, 128)
        pltpu.async_copy(rows[b], out_hbm.at[pl.ds(start, CHUNK), :], ssem[b])

    def drain_scatter(b):
        pltpu.make_async_copy(
            rows[b], out_hbm.at[pl.ds(0, CHUNK), :], ssem[b]).wait()

    fire_ef(0, 0)
    drain_ef(0)
    compute_codes(0, 0)
    fire_ef(1, 1)
    fire_gathers(0)
    drain_ef(1)
    fire_ef(2, 0)
    compute_codes(1, 1)
    fire_gathers(1)
    drain_gathers(0)
    fire_scatter(0, 0)

    def pair_body(i, carry):
        for k in (0, 1):
            ci = 2 * i + 1 + k
            cur = (1 + k) % 2
            nxt = 1 - cur
            drain_ef(nxt)
            compute_codes(nxt, nxt)

            @pl.when(ci + 2 < NCHUNK)
            def _prefetch():
                fire_ef(ci + 2, cur)

            drain_scatter(nxt)
            fire_gathers(nxt)
            drain_gathers(cur)
            fire_scatter(ci, cur)
        return carry

    lax.fori_loop(0, (NCHUNK - 2) // 2, pair_body, 0)
    drain_scatter(0)
    drain_gathers(1)
    fire_scatter(NCHUNK - 1, 1)
    drain_scatter(1)

    @pl.when(wid < XTRA)
    def _extra():
        estart = pl.multiple_of(wbase + BPW * 128, 128)
        pltpu.async_copy(ef_hbm.at[:, pl.ds(estart, 128)],
                         efb[0].at[:, pl.ds(0, 128)], efsem)
        pltpu.make_async_copy(
            ef_hbm.at[:, pl.ds(0, 128)],
            efb[0].at[:, pl.ds(0, 128)], efsem).wait()
        zero_i = jnp.zeros((16,), jnp.int32)
        for g in range(8):
            s16 = pl.ds(g * 16, 16)
            i0 = efb[0][0, s16]
            i1 = efb[0][1, s16]
            i2 = efb[0][2, s16]
            codes[0][g // GPS, pl.ds((g % GPS) * 16, 16)] = (
                (i0 * 6 + i1) * 2 + i2)
        for g in range(8, 2 * GPS):
            codes[0][g // GPS, pl.ds((g % GPS) * 16, 16)] = zero_i
        for j in range(2):
            pltpu.async_copy(combo_sp.at[codes[0].at[j]],
                             rows[0].at[pl.ds(j * SUB, SUB), :], gsem[0])
        pltpu.make_async_copy(
            out_hbm.at[pl.ds(0, 2 * SUB), :],
            rows[0].at[pl.ds(0, 2 * SUB), :], gsem[0]).wait()
        pltpu.sync_copy(rows[0].at[pl.ds(0, 128), :],
                        out_hbm.at[pl.ds(estart, 128), :])


def kernel(edge_feature, W0, W1, W2):
    ef_packed = edge_feature.T
    mesh = plsc.VectorSubcoreMesh(core_axis_name="c", subcore_axis_name="s")
    f = pl.kernel(
        _edge_encoder_body,
        out_type=jax.ShapeDtypeStruct((E, D), jnp.float32),
        mesh=mesh,
        scratch_types=[
            pltpu.VMEM((5, D), jnp.float32),
            pltpu.VMEM((6, D), jnp.float32),
            pltpu.VMEM((2, D), jnp.float32),
            pltpu.VMEM((NROW, D), jnp.float32),
            pltpu.VMEM_SHARED((NROW, D), jnp.float32),
            pltpu.VMEM((3, CHUNK), jnp.int32),
            pltpu.VMEM((3, CHUNK), jnp.int32),
            pltpu.VMEM((NSUB, SUB), jnp.int32),
            pltpu.VMEM((NSUB, SUB), jnp.int32),
            pltpu.VMEM((CHUNK, D), jnp.float32),
            pltpu.VMEM((CHUNK, D), jnp.float32),
            pltpu.SemaphoreType.DMA,
            pltpu.SemaphoreType.DMA,
            pltpu.SemaphoreType.DMA,
            pltpu.SemaphoreType.DMA,
            pltpu.SemaphoreType.DMA,
        ],
    )
    return f(ef_packed, W0, W1, W2)
